# baseline (speedup 1.0000x reference)
import functools

import jax
import jax.numpy as jnp
from jax import lax
from jax.experimental import pallas as pl
from jax.experimental.pallas import tpu as pltpu
from jax.experimental.pallas import tpu_sc as plsc

_B = 4096
_D = 256
_K = 8192
_TEMP = 0.05
_LOG2E = 1.4426950408889634
_LN2 = 0.6931471805599453

_BB = 1024
_BC = 8192

_NC = 2
_NS = 16
_NW = _NC * _NS
_BPW = _B // _NW


def _lse_kernel(x_ref, f_ref, a_ref, s_ref):
    x = x_ref[:, :]
    norm = jnp.maximum(jnp.sqrt(jnp.sum(x * x, axis=1, keepdims=True)), 1e-12)
    xn = (x * (_LOG2E / (norm * _TEMP))).astype(jnp.bfloat16)

    n_chunks = (2 * _K) // _BC
    half = n_chunks // 2
    ones = jnp.ones((_BC, 1), jnp.bfloat16)

    def body(ci, carry):
        sm, sh = carry
        chunk = f_ref[pl.ds(ci * _BC, _BC), :].astype(jnp.bfloat16)
        logits = jax.lax.dot_general(
            xn, chunk,
            dimension_numbers=(((1,), (1,)), ((), ())),
            preferred_element_type=jnp.float32,
            precision=jax.lax.Precision.DEFAULT,
        )
        e = jnp.exp2(logits.astype(jnp.bfloat16))
        s = jax.lax.dot_general(
            e, ones,
            dimension_numbers=(((1,), (0,)), ((), ())),
            preferred_element_type=jnp.float32,
            precision=jax.lax.Precision.DEFAULT,
        )[:, 0]
        is_mean = ci < half
        sm = jnp.where(is_mean, sm + s, sm)
        sh = jnp.where(is_mean, sh, sh + s)
        return sm, sh

    z = jnp.zeros((_BB,), jnp.float32)
    sm, sh = jax.lax.fori_loop(0, n_chunks, body, (z, z))

    a_ref[:, 0] = (0.5 * _LN2) * (jnp.log2(sm) + jnp.log2(sh))
    s_ref[:, 0] = 0.5 / (norm[:, 0] * _TEMP)


def _target_dot_kernel(x_hbm, t_hbm, f_hbm, dm_hbm, dh_hbm,
                       idx_v, idx2_v, x_v, rm_v, rh_v, dm_v, dh_v, sem):
    wid = lax.axis_index("s") * _NC + lax.axis_index("c")
    base = wid * _BPW

    pltpu.sync_copy(t_hbm.at[pl.ds(base, _BPW)], idx_v)
    pltpu.sync_copy(x_hbm.at[pl.ds(base, _BPW), :], x_v)
    for i in range(_BPW // 16):
        idx2_v[pl.ds(i * 16, 16)] = idx_v[pl.ds(i * 16, 16)] + _K
    pltpu.async_copy(f_hbm.at[idx_v], rm_v, sem).wait()
    pltpu.async_copy(f_hbm.at[idx2_v], rh_v, sem).wait()

    def row(r, c):
        zm = jnp.zeros((16,), jnp.float32)
        zh = jnp.zeros((16,), jnp.float32)
        for j in range(_D // 16):
            xx = x_v[r, pl.ds(j * 16, 16)]
            zm = zm + xx * rm_v[r, pl.ds(j * 16, 16)]
            zh = zh + xx * rh_v[r, pl.ds(j * 16, 16)]
        dm_v[pl.ds(r * 16, 16)] = zm
        dh_v[pl.ds(r * 16, 16)] = zh
        return c

    lax.fori_loop(0, _BPW, row, 0)
    pltpu.sync_copy(dm_v, dm_hbm.at[wid])
    pltpu.sync_copy(dh_v, dh_hbm.at[wid])


_sc_target_dots = functools.partial(
    pl.kernel,
    out_type=[
        jax.ShapeDtypeStruct((_NW, _BPW * 16), jnp.float32),
        jax.ShapeDtypeStruct((_NW, _BPW * 16), jnp.float32),
    ],
    mesh=plsc.VectorSubcoreMesh(core_axis_name="c", subcore_axis_name="s"),
    scratch_types=[
        pltpu.VMEM((_BPW,), jnp.int32),
        pltpu.VMEM((_BPW,), jnp.int32),
        pltpu.VMEM((_BPW, _D), jnp.float32),
        pltpu.VMEM((_BPW, _D), jnp.float32),
        pltpu.VMEM((_BPW, _D), jnp.float32),
        pltpu.VMEM((_BPW * 16,), jnp.float32),
        pltpu.VMEM((_BPW * 16,), jnp.float32),
        pltpu.SemaphoreType.DMA,
    ],
)(_target_dot_kernel)


@jax.jit
def _run(inputs, targets, features):
    t32 = targets.astype(jnp.int32)
    dm, dh = _sc_target_dots(inputs, t32, features)

    nb = _B // _BB
    a, s = pl.pallas_call(
        _lse_kernel,
        grid=(nb,),
        in_specs=[
            pl.BlockSpec((_BB, _D), lambda b: (b, 0)),
            pl.BlockSpec((2 * _K, _D), lambda b: (0, 0)),
        ],
        out_specs=[
            pl.BlockSpec((_BB, 1), lambda b: (b, 0)),
            pl.BlockSpec((_BB, 1), lambda b: (b, 0)),
        ],
        out_shape=[
            jax.ShapeDtypeStruct((_B, 1), jnp.float32),
            jax.ShapeDtypeStruct((_B, 1), jnp.float32),
        ],
    )(inputs, features)

    tgt = jnp.sum((dm + dh).reshape(_B, 16), axis=1)
    return jnp.mean(a[:, 0] - s[:, 0] * tgt)


def kernel(inputs, targets, features):
    return _run(inputs, targets, features)


# speedup vs baseline: 1.4552x; 1.4552x over previous
import functools

import jax
import jax.numpy as jnp
from jax import lax
from jax.experimental import pallas as pl
from jax.experimental.pallas import tpu as pltpu
from jax.experimental.pallas import tpu_sc as plsc

_B = 4096
_D = 256
_K = 8192
_TEMP = 0.05
_LOG2E = 1.4426950408889634
_LN2 = 0.6931471805599453

_BB = 2048
_BC = 4096

_NC = 2
_NS = 16
_NW = _NC * _NS
_BPW = _B // _NW


def _lse_kernel(x_ref, f_ref, a_ref, s_ref):
    x = x_ref[:, :]
    norm = jnp.maximum(jnp.sqrt(jnp.sum(x * x, axis=1, keepdims=True)), 1e-12)
    xn = x * (_LOG2E / (norm * _TEMP))

    n_chunks = (2 * _K) // _BC
    half = n_chunks // 2

    def body(ci, carry):
        sm, sh = carry
        chunk = f_ref[pl.ds(ci * _BC, _BC), :]
        logits = jax.lax.dot_general(
            xn, chunk,
            dimension_numbers=(((1,), (1,)), ((), ())),
            preferred_element_type=jnp.float32,
            precision=jax.lax.Precision.DEFAULT,
        )
        s = jnp.sum(jnp.exp2(logits), axis=1)
        is_mean = ci < half
        sm = jnp.where(is_mean, sm + s, sm)
        sh = jnp.where(is_mean, sh, sh + s)
        return sm, sh

    z = jnp.zeros((_BB,), jnp.float32)
    sm, sh = jax.lax.fori_loop(0, n_chunks, body, (z, z))

    a_ref[:, 0] = (0.5 * _LN2) * (jnp.log2(sm) + jnp.log2(sh))
    s_ref[:, 0] = 0.5 / (norm[:, 0] * _TEMP)


def _target_dot_kernel(x_hbm, t_hbm, f_hbm, dm_hbm, dh_hbm,
                       idx_v, idx2_v, x_v, rm_v, rh_v, dm_v, dh_v, sem):
    wid = lax.axis_index("s") * _NC + lax.axis_index("c")
    base = wid * _BPW

    pltpu.sync_copy(t_hbm.at[pl.ds(base, _BPW)], idx_v)
    pltpu.sync_copy(x_hbm.at[pl.ds(base, _BPW), :], x_v)
    for i in range(_BPW // 16):
        idx2_v[pl.ds(i * 16, 16)] = idx_v[pl.ds(i * 16, 16)] + _K
    pltpu.async_copy(f_hbm.at[idx_v], rm_v, sem).wait()
    pltpu.async_copy(f_hbm.at[idx2_v], rh_v, sem).wait()

    def row(r, c):
        zm = jnp.zeros((16,), jnp.float32)
        zh = jnp.zeros((16,), jnp.float32)
        for j in range(_D // 16):
            xx = x_v[r, pl.ds(j * 16, 16)]
            zm = zm + xx * rm_v[r, pl.ds(j * 16, 16)]
            zh = zh + xx * rh_v[r, pl.ds(j * 16, 16)]
        dm_v[pl.ds(r * 16, 16)] = zm
        dh_v[pl.ds(r * 16, 16)] = zh
        return c

    lax.fori_loop(0, _BPW, row, 0)
    pltpu.sync_copy(dm_v, dm_hbm.at[wid])
    pltpu.sync_copy(dh_v, dh_hbm.at[wid])


_sc_target_dots = functools.partial(
    pl.kernel,
    out_type=[
        jax.ShapeDtypeStruct((_NW, _BPW * 16), jnp.float32),
        jax.ShapeDtypeStruct((_NW, _BPW * 16), jnp.float32),
    ],
    mesh=plsc.VectorSubcoreMesh(core_axis_name="c", subcore_axis_name="s"),
    scratch_types=[
        pltpu.VMEM((_BPW,), jnp.int32),
        pltpu.VMEM((_BPW,), jnp.int32),
        pltpu.VMEM((_BPW, _D), jnp.float32),
        pltpu.VMEM((_BPW, _D), jnp.float32),
        pltpu.VMEM((_BPW, _D), jnp.float32),
        pltpu.VMEM((_BPW * 16,), jnp.float32),
        pltpu.VMEM((_BPW * 16,), jnp.float32),
        pltpu.SemaphoreType.DMA,
    ],
)(_target_dot_kernel)


@jax.jit
def _run(inputs, targets, features):
    t32 = targets.astype(jnp.int32)
    dm, dh = _sc_target_dots(inputs, t32, features)

    nb = _B // _BB
    a, s = pl.pallas_call(
        _lse_kernel,
        grid=(nb,),
        in_specs=[
            pl.BlockSpec((_BB, _D), lambda b: (b, 0)),
            pl.BlockSpec((2 * _K, _D), lambda b: (0, 0)),
        ],
        out_specs=[
            pl.BlockSpec((_BB, 1), lambda b: (b, 0)),
            pl.BlockSpec((_BB, 1), lambda b: (b, 0)),
        ],
        out_shape=[
            jax.ShapeDtypeStruct((_B, 1), jnp.float32),
            jax.ShapeDtypeStruct((_B, 1), jnp.float32),
        ],
    )(inputs, features)

    tgt = jnp.sum((dm + dh).reshape(_B, 16), axis=1)
    return jnp.mean(a[:, 0] - s[:, 0] * tgt)


def kernel(inputs, targets, features):
    return _run(inputs, targets, features)


# speedup vs baseline: 1.4925x; 1.0256x over previous
import functools

import jax
import jax.numpy as jnp
from jax import lax
from jax.experimental import pallas as pl
from jax.experimental.pallas import tpu as pltpu
from jax.experimental.pallas import tpu_sc as plsc

_B = 4096
_D = 256
_K = 8192
_TEMP = 0.05
_LOG2E = 1.4426950408889634
_LN2 = 0.6931471805599453

_BB = 1024
_BC = 8192

_NC = 2
_NS = 16
_NW = _NC * _NS
_BPW = _B // _NW


def _lse_kernel(x_ref, f_ref, a_ref, s_ref):
    x = x_ref[:, :]
    norm = jnp.maximum(jnp.sqrt(jnp.sum(x * x, axis=1, keepdims=True)), 1e-12)
    xn = x * (_LOG2E / (norm * _TEMP))

    n_chunks = (2 * _K) // _BC
    half = n_chunks // 2

    def body(ci, carry):
        sm, sh = carry
        chunk = f_ref[pl.ds(ci * _BC, _BC), :]
        logits = jax.lax.dot_general(
            xn, chunk,
            dimension_numbers=(((1,), (1,)), ((), ())),
            preferred_element_type=jnp.float32,
            precision=jax.lax.Precision.DEFAULT,
        )
        s = jnp.sum(jnp.exp2(logits), axis=1)
        is_mean = ci < half
        sm = jnp.where(is_mean, sm + s, sm)
        sh = jnp.where(is_mean, sh, sh + s)
        return sm, sh

    z = jnp.zeros((_BB,), jnp.float32)
    sm, sh = jax.lax.fori_loop(0, n_chunks, body, (z, z))

    a_ref[:, 0] = (0.5 * _LN2) * (jnp.log2(sm) + jnp.log2(sh))
    s_ref[:, 0] = 0.5 / (norm[:, 0] * _TEMP)


def _target_dot_kernel(x_hbm, t_hbm, f_hbm, dm_hbm, dh_hbm,
                       idx_v, idx2_v, x_v, rm_v, rh_v, dm_v, dh_v, sem):
    wid = lax.axis_index("s") * _NC + lax.axis_index("c")
    base = wid * _BPW

    pltpu.sync_copy(t_hbm.at[pl.ds(base, _BPW)], idx_v)
    pltpu.sync_copy(x_hbm.at[pl.ds(base, _BPW), :], x_v)
    for i in range(_BPW // 16):
        idx2_v[pl.ds(i * 16, 16)] = idx_v[pl.ds(i * 16, 16)] + _K
    pltpu.async_copy(f_hbm.at[idx_v], rm_v, sem).wait()
    pltpu.async_copy(f_hbm.at[idx2_v], rh_v, sem).wait()

    def row(r, c):
        zm = jnp.zeros((16,), jnp.float32)
        zh = jnp.zeros((16,), jnp.float32)
        for j in range(_D // 16):
            xx = x_v[r, pl.ds(j * 16, 16)]
            zm = zm + xx * rm_v[r, pl.ds(j * 16, 16)]
            zh = zh + xx * rh_v[r, pl.ds(j * 16, 16)]
        dm_v[pl.ds(r * 16, 16)] = zm
        dh_v[pl.ds(r * 16, 16)] = zh
        return c

    lax.fori_loop(0, _BPW, row, 0)
    pltpu.sync_copy(dm_v, dm_hbm.at[wid])
    pltpu.sync_copy(dh_v, dh_hbm.at[wid])


_sc_target_dots = functools.partial(
    pl.kernel,
    out_type=[
        jax.ShapeDtypeStruct((_NW, _BPW * 16), jnp.float32),
        jax.ShapeDtypeStruct((_NW, _BPW * 16), jnp.float32),
    ],
    mesh=plsc.VectorSubcoreMesh(core_axis_name="c", subcore_axis_name="s"),
    scratch_types=[
        pltpu.VMEM((_BPW,), jnp.int32),
        pltpu.VMEM((_BPW,), jnp.int32),
        pltpu.VMEM((_BPW, _D), jnp.float32),
        pltpu.VMEM((_BPW, _D), jnp.float32),
        pltpu.VMEM((_BPW, _D), jnp.float32),
        pltpu.VMEM((_BPW * 16,), jnp.float32),
        pltpu.VMEM((_BPW * 16,), jnp.float32),
        pltpu.SemaphoreType.DMA,
    ],
)(_target_dot_kernel)


@jax.jit
def _run(inputs, targets, features):
    t32 = targets.astype(jnp.int32)
    dm, dh = _sc_target_dots(inputs, t32, features)

    nb = _B // _BB
    a, s = pl.pallas_call(
        _lse_kernel,
        grid=(nb,),
        in_specs=[
            pl.BlockSpec((_BB, _D), lambda b: (b, 0)),
            pl.BlockSpec((2 * _K, _D), lambda b: (0, 0)),
        ],
        out_specs=[
            pl.BlockSpec((_BB, 1), lambda b: (b, 0)),
            pl.BlockSpec((_BB, 1), lambda b: (b, 0)),
        ],
        out_shape=[
            jax.ShapeDtypeStruct((_B, 1), jnp.float32),
            jax.ShapeDtypeStruct((_B, 1), jnp.float32),
        ],
    )(inputs, features)

    tgt = jnp.sum((dm + dh).reshape(_B, 16), axis=1)
    return jnp.mean(a[:, 0] - s[:, 0] * tgt)


def kernel(inputs, targets, features):
    return _run(inputs, targets, features)


# speedup vs baseline: 1.4935x; 1.0007x over previous
import functools

import jax
import jax.numpy as jnp
from jax import lax
from jax.experimental import pallas as pl
from jax.experimental.pallas import tpu as pltpu
from jax.experimental.pallas import tpu_sc as plsc

_B = 4096
_D = 256
_K = 8192
_TEMP = 0.05
_LOG2E = 1.4426950408889634
_LN2 = 0.6931471805599453

_BB = 1024
_BC = 8192

_NC = 2
_NS = 16
_NW = _NC * _NS
_BPW = _B // _NW


def _lse_kernel(x_ref, f_ref, a_ref, s_ref):
    x = x_ref[:, :]
    norm = jnp.maximum(jnp.sqrt(jnp.sum(x * x, axis=1, keepdims=True)), 1e-12)
    xn = x * (_LOG2E / (norm * _TEMP))

    n_chunks = (2 * _K) // _BC
    half = n_chunks // 2

    def body(ci, carry):
        sm, sh = carry
        chunk = f_ref[pl.ds(ci * _BC, _BC), :]
        logits = jax.lax.dot_general(
            xn, chunk,
            dimension_numbers=(((1,), (1,)), ((), ())),
            preferred_element_type=jnp.float32,
            precision=jax.lax.Precision.DEFAULT,
        )
        s = jnp.sum(jnp.exp2(logits), axis=1)
        is_mean = ci < half
        sm = jnp.where(is_mean, sm + s, sm)
        sh = jnp.where(is_mean, sh, sh + s)
        return sm, sh

    z = jnp.zeros((_BB,), jnp.float32)
    sm, sh = jax.lax.fori_loop(0, n_chunks, body, (z, z))

    a_ref[:, 0] = (0.5 * _LN2) * (jnp.log2(sm) + jnp.log2(sh))
    s_ref[:, 0] = 0.5 / (norm[:, 0] * _TEMP)


def _target_dot_kernel(x_hbm, t_hbm, f_hbm, dm_hbm, dh_hbm,
                       idx_v, idx2_v, x_v, rm_v, rh_v, dm_v, dh_v, sem):
    wid = lax.axis_index("s") * _NC + lax.axis_index("c")
    base = wid * _BPW

    pltpu.sync_copy(t_hbm.at[pl.ds(base, _BPW)], idx_v)
    pltpu.sync_copy(x_hbm.at[pl.ds(base, _BPW), :], x_v)
    for i in range(_BPW // 16):
        idx2_v[pl.ds(i * 16, 16)] = idx_v[pl.ds(i * 16, 16)] + _K
    cm = pltpu.async_copy(f_hbm.at[idx_v], rm_v, sem)
    ch = pltpu.async_copy(f_hbm.at[idx2_v], rh_v, sem)
    cm.wait()
    ch.wait()

    def row(r, c):
        zm = jnp.zeros((16,), jnp.float32)
        zh = jnp.zeros((16,), jnp.float32)
        for j in range(_D // 16):
            xx = x_v[r, pl.ds(j * 16, 16)]
            zm = zm + xx * rm_v[r, pl.ds(j * 16, 16)]
            zh = zh + xx * rh_v[r, pl.ds(j * 16, 16)]
        dm_v[pl.ds(r * 16, 16)] = zm
        dh_v[pl.ds(r * 16, 16)] = zh
        return c

    lax.fori_loop(0, _BPW, row, 0)
    pltpu.sync_copy(dm_v, dm_hbm.at[wid])
    pltpu.sync_copy(dh_v, dh_hbm.at[wid])


_sc_target_dots = functools.partial(
    pl.kernel,
    out_type=[
        jax.ShapeDtypeStruct((_NW, _BPW * 16), jnp.float32),
        jax.ShapeDtypeStruct((_NW, _BPW * 16), jnp.float32),
    ],
    mesh=plsc.VectorSubcoreMesh(core_axis_name="c", subcore_axis_name="s"),
    scratch_types=[
        pltpu.VMEM((_BPW,), jnp.int32),
        pltpu.VMEM((_BPW,), jnp.int32),
        pltpu.VMEM((_BPW, _D), jnp.float32),
        pltpu.VMEM((_BPW, _D), jnp.float32),
        pltpu.VMEM((_BPW, _D), jnp.float32),
        pltpu.VMEM((_BPW * 16,), jnp.float32),
        pltpu.VMEM((_BPW * 16,), jnp.float32),
        pltpu.SemaphoreType.DMA,
    ],
)(_target_dot_kernel)


@jax.jit
def _run(inputs, targets, features):
    t32 = targets.astype(jnp.int32)
    nb = _B // _BB
    a, s = pl.pallas_call(
        _lse_kernel,
        grid=(nb,),
        in_specs=[
            pl.BlockSpec((_BB, _D), lambda b: (b, 0)),
            pl.BlockSpec((2 * _K, _D), lambda b: (0, 0)),
        ],
        out_specs=[
            pl.BlockSpec((_BB, 1), lambda b: (b, 0)),
            pl.BlockSpec((_BB, 1), lambda b: (b, 0)),
        ],
        out_shape=[
            jax.ShapeDtypeStruct((_B, 1), jnp.float32),
            jax.ShapeDtypeStruct((_B, 1), jnp.float32),
        ],
    )(inputs, features)

    dm, dh = _sc_target_dots(inputs, t32, features)
    tgt = jnp.sum((dm + dh).reshape(_B, 16), axis=1)
    return jnp.mean(a[:, 0] - s[:, 0] * tgt)


def kernel(inputs, targets, features):
    return _run(inputs, targets, features)


# speedup vs baseline: 1.4969x; 1.0023x over previous
import functools

import jax
import jax.numpy as jnp
from jax import lax
from jax.experimental import pallas as pl
from jax.experimental.pallas import tpu as pltpu
from jax.experimental.pallas import tpu_sc as plsc

_B = 4096
_D = 256
_K = 8192
_TEMP = 0.05
_LOG2E = 1.4426950408889634
_LN2 = 0.6931471805599453

_BB = 1024
_BC = 8192

_NC = 2
_NS = 16
_NW = _NC * _NS
_BPW = _B // _NW


def _lse_kernel(x_ref, f_ref, a_ref, s_ref):
    x = x_ref[:, :]
    norm = jnp.maximum(jnp.sqrt(jnp.sum(x * x, axis=1, keepdims=True)), 1e-12)
    xn = x * (_LOG2E / (norm * _TEMP))

    n_chunks = (2 * _K) // _BC
    half = n_chunks // 2

    def body(ci, carry):
        sm, sh = carry
        chunk = f_ref[pl.ds(ci * _BC, _BC), :]
        logits = jax.lax.dot_general(
            xn, chunk,
            dimension_numbers=(((1,), (1,)), ((), ())),
            preferred_element_type=jnp.float32,
            precision=jax.lax.Precision.DEFAULT,
        )
        s = jnp.sum(jnp.exp2(logits), axis=1)
        is_mean = ci < half
        sm = jnp.where(is_mean, sm + s, sm)
        sh = jnp.where(is_mean, sh, sh + s)
        return sm, sh

    z = jnp.zeros((_BB,), jnp.float32)
    sm, sh = jax.lax.fori_loop(0, n_chunks, body, (z, z))

    a_ref[:, 0] = (0.5 * _LN2) * (jnp.log2(sm) + jnp.log2(sh))
    s_ref[:, 0] = 0.5 / (norm[:, 0] * _TEMP)


def _target_dot_kernel(x_hbm, t_hbm, f_hbm, d_hbm,
                       idx_v, idx2_v, x_v, rm_v, rh_v, d_v, sem):
    wid = lax.axis_index("s") * _NC + lax.axis_index("c")
    base = wid * _BPW

    pltpu.sync_copy(t_hbm.at[pl.ds(base, _BPW)], idx_v)
    pltpu.sync_copy(x_hbm.at[pl.ds(base, _BPW), :], x_v)
    for i in range(_BPW // 16):
        idx2_v[pl.ds(i * 16, 16)] = idx_v[pl.ds(i * 16, 16)] + _K
    cm = pltpu.async_copy(f_hbm.at[idx_v], rm_v, sem)
    ch = pltpu.async_copy(f_hbm.at[idx2_v], rh_v, sem)
    cm.wait()
    ch.wait()

    def row(r, c):
        zm = jnp.zeros((16,), jnp.float32)
        zh = jnp.zeros((16,), jnp.float32)
        for j in range(_D // 16):
            xx = x_v[r, pl.ds(j * 16, 16)]
            zm = zm + xx * rm_v[r, pl.ds(j * 16, 16)]
            zh = zh + xx * rh_v[r, pl.ds(j * 16, 16)]
        d_v[pl.ds(r * 16, 16)] = zm + zh
        return c

    lax.fori_loop(0, _BPW, row, 0)
    pltpu.sync_copy(d_v, d_hbm.at[wid])


_sc_target_dots = functools.partial(
    pl.kernel,
    out_type=jax.ShapeDtypeStruct((_NW, _BPW * 16), jnp.float32),
    mesh=plsc.VectorSubcoreMesh(core_axis_name="c", subcore_axis_name="s"),
    scratch_types=[
        pltpu.VMEM((_BPW,), jnp.int32),
        pltpu.VMEM((_BPW,), jnp.int32),
        pltpu.VMEM((_BPW, _D), jnp.float32),
        pltpu.VMEM((_BPW, _D), jnp.float32),
        pltpu.VMEM((_BPW, _D), jnp.float32),
        pltpu.VMEM((_BPW * 16,), jnp.float32),
        pltpu.SemaphoreType.DMA,
    ],
)(_target_dot_kernel)


@jax.jit
def _run(inputs, targets, features):
    t32 = targets.astype(jnp.int32)
    nb = _B // _BB
    a, s = pl.pallas_call(
        _lse_kernel,
        grid=(nb,),
        in_specs=[
            pl.BlockSpec((_BB, _D), lambda b: (b, 0)),
            pl.BlockSpec((2 * _K, _D), lambda b: (0, 0)),
        ],
        out_specs=[
            pl.BlockSpec((_BB, 1), lambda b: (b, 0)),
            pl.BlockSpec((_BB, 1), lambda b: (b, 0)),
        ],
        out_shape=[
            jax.ShapeDtypeStruct((_B, 1), jnp.float32),
            jax.ShapeDtypeStruct((_B, 1), jnp.float32),
        ],
    )(inputs, features)

    d = _sc_target_dots(inputs, t32, features)
    tgt = jnp.sum(d.reshape(_B, 16), axis=1)
    return jnp.mean(a[:, 0] - s[:, 0] * tgt)


def kernel(inputs, targets, features):
    return _run(inputs, targets, features)


# speedup vs baseline: 1.5739x; 1.0514x over previous
import functools

import jax
import jax.numpy as jnp
from jax import lax
from jax.experimental import pallas as pl
from jax.experimental.pallas import tpu as pltpu
from jax.experimental.pallas import tpu_sc as plsc

_B = 4096
_D = 256
_K = 8192
_TEMP = 0.05
_LOG2E = 1.4426950408889634
_LN2 = 0.6931471805599453

_BB = 1024
_BC = 8192

_NC = 2
_NS = 16
_NW = _NC * _NS
_BPW = _B // _NW


def _lse_kernel(x_ref, f_ref, a_ref, s_ref):
    x = x_ref[:, :]
    norm = jnp.maximum(jnp.sqrt(jnp.sum(x * x, axis=1, keepdims=True)), 1e-12)
    xn = x * (_LOG2E / (norm * _TEMP))

    def half_sum(ci):
        chunk = f_ref[pl.ds(ci * _BC, _BC), :]
        logits = jax.lax.dot_general(
            xn, chunk,
            dimension_numbers=(((1,), (1,)), ((), ())),
            preferred_element_type=jnp.float32,
            precision=jax.lax.Precision.DEFAULT,
        )
        return jnp.sum(jnp.exp2(logits), axis=1)

    sm = half_sum(0)
    sh = half_sum(1)

    a_ref[:, 0] = (0.5 * _LN2) * (jnp.log2(sm) + jnp.log2(sh))
    s_ref[:, 0] = 0.5 / (norm[:, 0] * _TEMP)


def _target_dot_kernel(x_hbm, t_hbm, f_hbm, d_hbm,
                       idx_v, idx2_v, x_v, rm_v, rh_v, d_v, sem):
    wid = lax.axis_index("s") * _NC + lax.axis_index("c")
    base = wid * _BPW

    pltpu.sync_copy(t_hbm.at[pl.ds(base, _BPW)], idx_v)
    pltpu.sync_copy(x_hbm.at[pl.ds(base, _BPW), :], x_v)
    for i in range(_BPW // 16):
        idx2_v[pl.ds(i * 16, 16)] = idx_v[pl.ds(i * 16, 16)] + _K
    cm = pltpu.async_copy(f_hbm.at[idx_v], rm_v, sem)
    ch = pltpu.async_copy(f_hbm.at[idx2_v], rh_v, sem)
    cm.wait()
    ch.wait()

    def row(r, c):
        zm = jnp.zeros((16,), jnp.float32)
        zh = jnp.zeros((16,), jnp.float32)
        for j in range(_D // 16):
            xx = x_v[r, pl.ds(j * 16, 16)]
            zm = zm + xx * rm_v[r, pl.ds(j * 16, 16)]
            zh = zh + xx * rh_v[r, pl.ds(j * 16, 16)]
        d_v[pl.ds(r * 16, 16)] = zm + zh
        return c

    lax.fori_loop(0, _BPW, row, 0)
    pltpu.sync_copy(d_v, d_hbm.at[wid])


_sc_target_dots = functools.partial(
    pl.kernel,
    out_type=jax.ShapeDtypeStruct((_NW, _BPW * 16), jnp.float32),
    mesh=plsc.VectorSubcoreMesh(core_axis_name="c", subcore_axis_name="s"),
    scratch_types=[
        pltpu.VMEM((_BPW,), jnp.int32),
        pltpu.VMEM((_BPW,), jnp.int32),
        pltpu.VMEM((_BPW, _D), jnp.float32),
        pltpu.VMEM((_BPW, _D), jnp.float32),
        pltpu.VMEM((_BPW, _D), jnp.float32),
        pltpu.VMEM((_BPW * 16,), jnp.float32),
        pltpu.SemaphoreType.DMA,
    ],
)(_target_dot_kernel)


@jax.jit
def _run(inputs, targets, features):
    t32 = targets.astype(jnp.int32)
    nb = _B // _BB
    a, s = pl.pallas_call(
        _lse_kernel,
        grid=(nb,),
        in_specs=[
            pl.BlockSpec((_BB, _D), lambda b: (b, 0)),
            pl.BlockSpec((2 * _K, _D), lambda b: (0, 0)),
        ],
        out_specs=[
            pl.BlockSpec((_BB, 1), lambda b: (b, 0)),
            pl.BlockSpec((_BB, 1), lambda b: (b, 0)),
        ],
        out_shape=[
            jax.ShapeDtypeStruct((_B, 1), jnp.float32),
            jax.ShapeDtypeStruct((_B, 1), jnp.float32),
        ],
    )(inputs, features)

    d = _sc_target_dots(inputs, t32, features)
    tgt = jnp.sum(d.reshape(_B, 16), axis=1)
    return jnp.mean(a[:, 0] - s[:, 0] * tgt)


def kernel(inputs, targets, features):
    return _run(inputs, targets, features)
